# Initial kernel scaffold; baseline (speedup 1.0000x reference)
import jax
import jax.numpy as jnp
from jax.experimental import pallas as pl


def kernel(queries, keys_mem, values_mem, access_counts, last_access, time_step, k):
    raise NotImplementedError("write your pallas kernel here")


# speedup vs baseline: 7.5597x; 7.5597x over previous
import functools

import jax
import jax.numpy as jnp
from jax.experimental import pallas as pl

_N = 100000
_D = 64
_B = 1024
_KTOP = 32
_MAXA = 10000.0
_KB = 2048
_NBLK = 49
_NP = _NBLK * _KB
_W = 128
_CPB = _KB // _W
_C = _NBLK * _CPB
_T = 40
_NEG = -3.0e38
_BIGI = 2**30


def _sims_body(q_ref, k_ref, s_ref, cm_ref):
    i = pl.program_id(0)
    q = q_ref[...]
    qs = jnp.sum(q * q, axis=1, keepdims=True)
    qn = q / jnp.maximum(jnp.sqrt(qs), 1e-8)
    kb = k_ref[...]
    ks = jnp.sum(kb * kb, axis=1, keepdims=True)
    kn = kb / jnp.maximum(jnp.sqrt(ks), 1e-8)
    s = jax.lax.dot_general(
        qn, kn, (((1,), (1,)), ((), ())),
        preferred_element_type=jnp.float32,
    )

    @pl.when(i < _NBLK - 1)
    def _():
        s_ref[...] = s
        cm_ref[...] = jnp.max(s.reshape(_B, _CPB, _W), axis=2)[None]

    @pl.when(i == _NBLK - 1)
    def _():
        lane = jax.lax.broadcasted_iota(jnp.int32, (_B, _KB), 1)
        col = i * _KB + lane
        s2 = jnp.where(col < _N, s, -2.0)
        s_ref[...] = s2
        cm_ref[...] = jnp.max(s2.reshape(_B, _CPB, _W), axis=2)[None]


def _run_sims(queries, keys_p):
    return pl.pallas_call(
        _sims_body,
        grid=(_NBLK,),
        in_specs=[
            pl.BlockSpec((_B, _D), lambda i: (0, 0)),
            pl.BlockSpec((_KB, _D), lambda i: (i, 0)),
        ],
        out_specs=[
            pl.BlockSpec((_B, _KB), lambda i: (0, i)),
            pl.BlockSpec((1, _B, _CPB), lambda i: (i, 0, 0)),
        ],
        out_shape=[
            jax.ShapeDtypeStruct((_B, _NP), jnp.float32),
            jax.ShapeDtypeStruct((_NBLK, _B, _CPB), jnp.float32),
        ],
    )(queries, keys_p)


def _chunksel_body(cmt_ref, ids_ref, flat_ref):
    a = cmt_ref[...]
    row = jax.lax.broadcasted_iota(jnp.int32, (_C, _B), 0)
    qlane = jax.lax.broadcasted_iota(jnp.int32, (1, _B), 1)
    ids = []
    for t in range(_T):
        m = jnp.max(a, axis=0, keepdims=True)
        sel = jnp.where(a >= m, row, _BIGI)
        idx = jnp.min(sel, axis=0, keepdims=True)
        ids.append(idx)
        a = jnp.where(row == idx, _NEG, a)
    ids = jnp.concatenate(ids, axis=0)
    ids_ref[...] = ids
    flat_ref[...] = qlane * _C + ids


def _run_chunksel(cm_t):
    return pl.pallas_call(
        _chunksel_body,
        in_specs=[pl.BlockSpec((_C, _B), lambda: (0, 0))],
        out_specs=[
            pl.BlockSpec((_T, _B), lambda: (0, 0)),
            pl.BlockSpec((_T, _B), lambda: (0, 0)),
        ],
        out_shape=[
            jax.ShapeDtypeStruct((_T, _B), jnp.int32),
            jax.ShapeDtypeStruct((_T, _B), jnp.int32),
        ],
    )(cm_t)


_RTILE = 128
_CW = _T * _W


def _final_body(cand_ref, ids_ref, topv_ref, topi_ref, w_ref):
    a = cand_ref[...]
    selids = ids_ref[...]
    lane = jax.lax.broadcasted_iota(jnp.int32, (_RTILE, _CW), 1)
    tlane = jax.lax.broadcasted_iota(jnp.int32, (_RTILE, _T), 1)
    vals, gids = [], []
    for r in range(_KTOP):
        m = jnp.max(a, axis=1, keepdims=True)
        sel = jnp.where(a >= m, lane, _BIGI)
        pos = jnp.min(sel, axis=1, keepdims=True)
        tstar = jax.lax.shift_right_logical(pos, 7)
        wstar = jnp.bitwise_and(pos, 127)
        cid = jnp.sum(
            jnp.where(tlane == tstar, selids, 0), axis=1, keepdims=True)
        gids.append(cid * _W + wstar)
        vals.append(m)
        a = jnp.where(lane == pos, _NEG, a)
    topv = jnp.concatenate(vals, axis=1)
    topi = jnp.concatenate(gids, axis=1)
    topv_ref[...] = topv
    topi_ref[...] = topi
    e = jnp.exp(topv - topv[:, :1])
    w_ref[...] = e / jnp.sum(e, axis=1, keepdims=True)


def _run_final(cand, selids_q):
    return pl.pallas_call(
        _final_body,
        grid=(_B // _RTILE,),
        in_specs=[
            pl.BlockSpec((_RTILE, _CW), lambda i: (i, 0)),
            pl.BlockSpec((_RTILE, _T), lambda i: (i, 0)),
        ],
        out_specs=[
            pl.BlockSpec((_RTILE, _KTOP), lambda i: (i, 0)),
            pl.BlockSpec((_RTILE, _KTOP), lambda i: (i, 0)),
            pl.BlockSpec((_RTILE, _KTOP), lambda i: (i, 0)),
        ],
        out_shape=[
            jax.ShapeDtypeStruct((_B, _KTOP), jnp.float32),
            jax.ShapeDtypeStruct((_B, _KTOP), jnp.int32),
            jax.ShapeDtypeStruct((_B, _KTOP), jnp.float32),
        ],
    )(cand, selids_q)


def _combine_body(v_ref, w_ref, o_ref):
    v = v_ref[...]
    w = w_ref[...]
    acc = jnp.zeros((_B, _D), jnp.float32)
    for t in range(_KTOP):
        acc = acc + v[:, t * _D:(t + 1) * _D] * w[:, t:t + 1]
    o_ref[...] = acc


def _run_combine(vals_g, weights):
    return pl.pallas_call(
        _combine_body,
        in_specs=[
            pl.BlockSpec((_B, _KTOP * _D), lambda: (0, 0)),
            pl.BlockSpec((_B, _KTOP), lambda: (0, 0)),
        ],
        out_specs=pl.BlockSpec((_B, _D), lambda: (0, 0)),
        out_shape=jax.ShapeDtypeStruct((_B, _D), jnp.float32),
    )(vals_g, weights)


_HR = 98


def _stats_body(h_ref, c_ref, l_ref, t_ref, co_ref, lo_ref):
    h = h_ref[...]
    c = c_ref[...]
    lst = l_ref[...]
    t = t_ref[...]
    co_ref[...] = jnp.minimum(c + h, _MAXA)
    lo_ref[...] = jnp.where(h > 0, t[0, 0], lst)


def _run_stats(hist2, counts2, last2, tarr):
    return pl.pallas_call(
        _stats_body,
        in_specs=[
            pl.BlockSpec((_HR, 1024), lambda: (0, 0)),
            pl.BlockSpec((_HR, 1024), lambda: (0, 0)),
            pl.BlockSpec((_HR, 1024), lambda: (0, 0)),
            pl.BlockSpec((8, 128), lambda: (0, 0)),
        ],
        out_specs=[
            pl.BlockSpec((_HR, 1024), lambda: (0, 0)),
            pl.BlockSpec((_HR, 1024), lambda: (0, 0)),
        ],
        out_shape=[
            jax.ShapeDtypeStruct((_HR, 1024), jnp.float32),
            jax.ShapeDtypeStruct((_HR, 1024), jnp.float32),
        ],
    )(hist2, counts2, last2, tarr)


def kernel(queries, keys_mem, values_mem, access_counts, last_access, time_step, k):
    keys_p = jnp.concatenate(
        [keys_mem, jnp.zeros((_NP - _N, _D), jnp.float32)], axis=0)
    sims, cm = _run_sims(queries, keys_p)

    cm_t = cm.transpose(0, 2, 1).reshape(_C, _B)
    selids, flatrows = _run_chunksel(cm_t)

    sims_flat = sims.reshape(_B * _C, _W)
    idx_q = flatrows.T.reshape(-1)
    cand = sims_flat[idx_q]
    cand = cand.reshape(_B, _CW)

    selids_q = selids.T
    topv, topi, weights = _run_final(cand, selids_q)

    vals_g = values_mem[topi.reshape(-1)]
    vals_g = vals_g.reshape(_B, _KTOP * _D)
    combined = _run_combine(vals_g, weights)

    hist = jnp.zeros((_NP,), jnp.float32).at[topi.reshape(-1)].add(1.0)

    counts2 = jnp.concatenate(
        [access_counts, jnp.zeros((_NP - _N,), jnp.float32)]).reshape(_HR, 1024)
    last2 = jnp.concatenate(
        [last_access, jnp.zeros((_NP - _N,), jnp.float32)]).reshape(_HR, 1024)
    tarr = jnp.full((8, 128), time_step, jnp.float32)
    co, lo = _run_stats(hist.reshape(_HR, 1024), counts2, last2, tarr)

    k_dep = (jnp.minimum(k, _N) - _KTOP).astype(jnp.float32)
    top_k_values = topv + k_dep
    new_access_counts = co.reshape(-1)[:_N]
    new_last_access = lo.reshape(-1)[:_N]
    return (combined, top_k_values, new_access_counts, new_last_access)


# speedup vs baseline: 8.3165x; 1.1001x over previous
import functools

import jax
import jax.numpy as jnp
from jax.experimental import pallas as pl
from jax.experimental.pallas import tpu as pltpu
from jax.experimental.pallas import tpu_sc as plsc

_N = 100000
_D = 64
_B = 1024
_KTOP = 32
_MAXA = 10000.0
_KB = 2048
_NBLK = 49
_NP = _NBLK * _KB
_W = 128
_CPB = _KB // _W
_C = _NBLK * _CPB
_T = 40
_NEG = -3.0e38
_BIGI = 2**30


def _sc_mesh():
    return plsc.VectorSubcoreMesh(core_axis_name="c", subcore_axis_name="s")


def _sc_gather(table, idx, width, window=128):
    num = idx.shape[0]
    idx2 = idx.reshape(1, num)

    @functools.partial(
        pl.kernel,
        out_type=jax.ShapeDtypeStruct((num, width), table.dtype),
        mesh=_sc_mesh(),
    )
    def _k(x_hbm, i_hbm, o_hbm):
        def body(i_vmem, o_vmem):
            pltpu.sync_copy(x_hbm.at[i_vmem.at[0]], o_vmem)

        pltpu.emit_pipeline(
            body,
            grid=(num // window,),
            in_specs=[pl.BlockSpec((1, window), index_map=lambda i: (0, i))],
            out_specs=[pl.BlockSpec((window, width),
                                    index_map=lambda i: (i, 0))],
            core_axis_name=("c", "s"),
            dimension_semantics=(pltpu.PARALLEL,),
        )(i_hbm, o_hbm)

    return _k(table, idx2)


def _sc_hist(idx32, zeros_col, ones_col):
    @functools.partial(
        pl.kernel,
        out_type=jax.ShapeDtypeStruct((2, _NP), jnp.float32),
        mesh=_sc_mesh(),
        scratch_types=[
            pltpu.VMEM((8, 128), jnp.int32),
            pltpu.VMEM((128,), jnp.float32),
            pltpu.VMEM_SHARED((_NP,), jnp.float32),
        ],
    )
    def _k(i_hbm, z_hbm, one_hbm, o_hbm, idx_v, ones_v, hist_sh):
        cid = jax.lax.axis_index("c")
        sid = jax.lax.axis_index("s")
        wid = cid * 16 + sid

        @pl.when(sid == 0)
        def _():
            pltpu.sync_copy(z_hbm, hist_sh)

        plsc.subcore_barrier()
        pltpu.sync_copy(i_hbm.at[wid], idx_v)
        pltpu.sync_copy(one_hbm, ones_v)
        for j in range(8):
            pltpu.sync_copy(ones_v, hist_sh.at[idx_v.at[j]], add=True)
        plsc.subcore_barrier()

        @pl.when(sid == 0)
        def _():
            pltpu.sync_copy(hist_sh, o_hbm.at[cid])

    return _k(idx32, zeros_col, ones_col)


def _sims_body(q_ref, k_ref, s_ref, cmt_ref):
    i = pl.program_id(0)
    q = q_ref[...]
    qs = jnp.sum(q * q, axis=1, keepdims=True)
    qn = q / jnp.maximum(jnp.sqrt(qs), 1e-8)
    kb = k_ref[...]
    ks = jnp.sum(kb * kb, axis=1, keepdims=True)
    kn = kb / jnp.maximum(jnp.sqrt(ks), 1e-8)
    s = jax.lax.dot_general(
        qn, kn, (((1,), (1,)), ((), ())),
        preferred_element_type=jnp.float32,
    )
    st = jax.lax.dot_general(
        kn, qn, (((1,), (1,)), ((), ())),
        preferred_element_type=jnp.float32,
    )

    @pl.when(i < _NBLK - 1)
    def _():
        s_ref[...] = s
        cmt_ref[...] = jnp.max(st.reshape(_CPB, _W, _B), axis=1)

    @pl.when(i == _NBLK - 1)
    def _():
        lane = jax.lax.broadcasted_iota(jnp.int32, (_B, _KB), 1)
        s_ref[...] = jnp.where(i * _KB + lane < _N, s, -2.0)
        row = jax.lax.broadcasted_iota(jnp.int32, (_KB, _B), 0)
        st2 = jnp.where(i * _KB + row < _N, st, -2.0)
        cmt_ref[...] = jnp.max(st2.reshape(_CPB, _W, _B), axis=1)


def _run_sims(queries, keys_p):
    return pl.pallas_call(
        _sims_body,
        grid=(_NBLK,),
        in_specs=[
            pl.BlockSpec((_B, _D), lambda i: (0, 0)),
            pl.BlockSpec((_KB, _D), lambda i: (i, 0)),
        ],
        out_specs=[
            pl.BlockSpec((_B, _KB), lambda i: (0, i)),
            pl.BlockSpec((_CPB, _B), lambda i: (i, 0)),
        ],
        out_shape=[
            jax.ShapeDtypeStruct((_B, _NP), jnp.float32),
            jax.ShapeDtypeStruct((_C, _B), jnp.float32),
        ],
    )(queries, keys_p)


def _chunksel_body(cmt_ref, ids_ref, flat_ref):
    a = cmt_ref[...]
    row = jax.lax.broadcasted_iota(jnp.int32, (_C, _B), 0)
    qlane = jax.lax.broadcasted_iota(jnp.int32, (1, _B), 1)
    ids = []
    for t in range(_T):
        m = jnp.max(a, axis=0, keepdims=True)
        sel = jnp.where(a >= m, row, _BIGI)
        idx = jnp.min(sel, axis=0, keepdims=True)
        ids.append(idx)
        a = jnp.where(row == idx, _NEG, a)
    ids = jnp.concatenate(ids, axis=0)
    ids_ref[...] = ids
    flat_ref[...] = qlane * _C + ids


def _run_chunksel(cm_t):
    return pl.pallas_call(
        _chunksel_body,
        in_specs=[pl.BlockSpec((_C, _B), lambda: (0, 0))],
        out_specs=[
            pl.BlockSpec((_T, _B), lambda: (0, 0)),
            pl.BlockSpec((_T, _B), lambda: (0, 0)),
        ],
        out_shape=[
            jax.ShapeDtypeStruct((_T, _B), jnp.int32),
            jax.ShapeDtypeStruct((_T, _B), jnp.int32),
        ],
    )(cm_t)


_RTILE = 128
_CW = _T * _W


def _final_body(cand_ref, ids_ref, topv_ref, topi_ref, w_ref):
    a = cand_ref[...]
    selids = ids_ref[...]
    lane = jax.lax.broadcasted_iota(jnp.int32, (_RTILE, _CW), 1)
    tlane = jax.lax.broadcasted_iota(jnp.int32, (_RTILE, _T), 1)
    vals, gids = [], []
    for r in range(_KTOP):
        m = jnp.max(a, axis=1, keepdims=True)
        sel = jnp.where(a >= m, lane, _BIGI)
        pos = jnp.min(sel, axis=1, keepdims=True)
        tstar = jax.lax.shift_right_logical(pos, 7)
        wstar = jnp.bitwise_and(pos, 127)
        cid = jnp.sum(
            jnp.where(tlane == tstar, selids, 0), axis=1, keepdims=True)
        gids.append(cid * _W + wstar)
        vals.append(m)
        a = jnp.where(lane == pos, _NEG, a)
    topv = jnp.concatenate(vals, axis=1)
    topi = jnp.concatenate(gids, axis=1)
    topv_ref[...] = topv
    topi_ref[...] = topi
    e = jnp.exp(topv - topv[:, :1])
    w_ref[...] = e / jnp.sum(e, axis=1, keepdims=True)


def _run_final(cand, selids_q):
    return pl.pallas_call(
        _final_body,
        grid=(_B // _RTILE,),
        in_specs=[
            pl.BlockSpec((_RTILE, _CW), lambda i: (i, 0)),
            pl.BlockSpec((_RTILE, _T), lambda i: (i, 0)),
        ],
        out_specs=[
            pl.BlockSpec((_RTILE, _KTOP), lambda i: (i, 0)),
            pl.BlockSpec((_RTILE, _KTOP), lambda i: (i, 0)),
            pl.BlockSpec((_RTILE, _KTOP), lambda i: (i, 0)),
        ],
        out_shape=[
            jax.ShapeDtypeStruct((_B, _KTOP), jnp.float32),
            jax.ShapeDtypeStruct((_B, _KTOP), jnp.int32),
            jax.ShapeDtypeStruct((_B, _KTOP), jnp.float32),
        ],
    )(cand, selids_q)


def _combine_body(v_ref, p_ref, w_ref, o_ref):
    v = v_ref[...]
    p = p_ref[...]
    w = w_ref[...]
    acc = jnp.zeros((_RTILE, _D), jnp.float32)
    for t in range(_KTOP):
        lo = v[:, t * 128:t * 128 + _D]
        hi = v[:, t * 128 + _D:(t + 1) * 128]
        sel = jnp.where(p[:, t:t + 1] == 1, hi, lo)
        acc = acc + sel * w[:, t:t + 1]
    o_ref[...] = acc


def _run_combine(vals_g, parity, weights):
    return pl.pallas_call(
        _combine_body,
        grid=(_B // _RTILE,),
        in_specs=[
            pl.BlockSpec((_RTILE, _KTOP * 128), lambda i: (i, 0)),
            pl.BlockSpec((_RTILE, _KTOP), lambda i: (i, 0)),
            pl.BlockSpec((_RTILE, _KTOP), lambda i: (i, 0)),
        ],
        out_specs=pl.BlockSpec((_RTILE, _D), lambda i: (i, 0)),
        out_shape=jax.ShapeDtypeStruct((_B, _D), jnp.float32),
    )(vals_g, parity, weights)


_HR = 98


def _stats_body(h_ref, c_ref, l_ref, t_ref, co_ref, lo_ref):
    h = h_ref[0] + h_ref[1]
    c = c_ref[...]
    lst = l_ref[...]
    t = t_ref[...]
    co_ref[...] = jnp.minimum(c + h, _MAXA)
    lo_ref[...] = jnp.where(h > 0, t[0, 0], lst)


def _run_stats(hist2, counts2, last2, tarr):
    return pl.pallas_call(
        _stats_body,
        in_specs=[
            pl.BlockSpec((2, _HR, 1024), lambda: (0, 0, 0)),
            pl.BlockSpec((_HR, 1024), lambda: (0, 0)),
            pl.BlockSpec((_HR, 1024), lambda: (0, 0)),
            pl.BlockSpec((8, 128), lambda: (0, 0)),
        ],
        out_specs=[
            pl.BlockSpec((_HR, 1024), lambda: (0, 0)),
            pl.BlockSpec((_HR, 1024), lambda: (0, 0)),
        ],
        out_shape=[
            jax.ShapeDtypeStruct((_HR, 1024), jnp.float32),
            jax.ShapeDtypeStruct((_HR, 1024), jnp.float32),
        ],
    )(hist2, counts2, last2, tarr)


def kernel(queries, keys_mem, values_mem, access_counts, last_access, time_step, k):
    keys_p = jnp.concatenate(
        [keys_mem, jnp.zeros((_NP - _N, _D), jnp.float32)], axis=0)
    sims, cm_t = _run_sims(queries, keys_p)
    selids, flatrows = _run_chunksel(cm_t)

    sims_flat = sims.reshape(_B * _C, _W)
    idx_q = flatrows.T.reshape(-1)
    cand = _sc_gather(sims_flat, idx_q, _W)
    cand = cand.reshape(_B, _CW)

    selids_q = selids.T
    topv, topi, weights = _run_final(cand, selids_q)

    pair_idx = jax.lax.shift_right_logical(topi, 1)
    parity = jnp.bitwise_and(topi, 1)
    vals_g = _sc_gather(
        values_mem.reshape(_N // 2, 2 * _D), pair_idx.reshape(-1), 2 * _D)
    vals_g = vals_g.reshape(_B, _KTOP * 2 * _D)
    combined = _run_combine(vals_g, parity, weights)

    hists = _sc_hist(
        topi.reshape(32, 8, 128),
        jnp.zeros((_NP,), jnp.float32),
        jnp.ones((128,), jnp.float32),
    )

    counts2 = jnp.concatenate(
        [access_counts, jnp.zeros((_NP - _N,), jnp.float32)]).reshape(_HR, 1024)
    last2 = jnp.concatenate(
        [last_access, jnp.zeros((_NP - _N,), jnp.float32)]).reshape(_HR, 1024)
    tarr = jnp.full((8, 128), time_step, jnp.float32)
    co, lo = _run_stats(
        hists.reshape(2, _HR, 1024), counts2, last2, tarr)

    k_dep = (jnp.minimum(k, _N) - _KTOP).astype(jnp.float32)
    top_k_values = topv + k_dep
    new_access_counts = co.reshape(-1)[:_N]
    new_last_access = lo.reshape(-1)[:_N]
    return (combined, top_k_values, new_access_counts, new_last_access)


# speedup vs baseline: 8.5361x; 1.0264x over previous
import functools

import jax
import jax.numpy as jnp
from jax.experimental import pallas as pl
from jax.experimental.pallas import tpu as pltpu
from jax.experimental.pallas import tpu_sc as plsc

_N = 100000
_D = 64
_B = 1024
_KTOP = 32
_MAXA = 10000.0
_KB = 2048
_NBLK = 49
_NP = _NBLK * _KB
_W = 128
_CPB = _KB // _W
_C = _NBLK * _CPB
_T = 40
_NEG = -3.0e38
_BIGI = 2**30


def _sc_mesh():
    return plsc.VectorSubcoreMesh(core_axis_name="c", subcore_axis_name="s")


def _sc_gather(table, idx, width, window=128):
    num = idx.shape[0]
    idx2 = idx.reshape(1, num)

    @functools.partial(
        pl.kernel,
        out_type=jax.ShapeDtypeStruct((num, width), table.dtype),
        mesh=_sc_mesh(),
    )
    def _k(x_hbm, i_hbm, o_hbm):
        def body(i_vmem, o_vmem):
            pltpu.sync_copy(x_hbm.at[i_vmem.at[0]], o_vmem)

        pltpu.emit_pipeline(
            body,
            grid=(num // window,),
            in_specs=[pl.BlockSpec((1, window), index_map=lambda i: (0, i))],
            out_specs=[pl.BlockSpec((window, width),
                                    index_map=lambda i: (i, 0))],
            core_axis_name=("c", "s"),
            dimension_semantics=(pltpu.PARALLEL,),
        )(i_hbm, o_hbm)

    return _k(table, idx2)


def _sc_hist(idx32, zeros_col, ones_col):
    @functools.partial(
        pl.kernel,
        out_type=jax.ShapeDtypeStruct((2, _NP), jnp.float32),
        mesh=_sc_mesh(),
        scratch_types=[
            pltpu.VMEM((8, 128), jnp.int32),
            pltpu.VMEM((128,), jnp.float32),
            pltpu.VMEM_SHARED((_NP,), jnp.float32),
        ],
    )
    def _k(i_hbm, z_hbm, one_hbm, o_hbm, idx_v, ones_v, hist_sh):
        cid = jax.lax.axis_index("c")
        sid = jax.lax.axis_index("s")
        wid = cid * 16 + sid

        @pl.when(sid == 0)
        def _():
            pltpu.sync_copy(z_hbm, hist_sh)

        plsc.subcore_barrier()
        pltpu.sync_copy(i_hbm.at[wid], idx_v)
        pltpu.sync_copy(one_hbm, ones_v)
        for j in range(8):
            pltpu.sync_copy(ones_v, hist_sh.at[idx_v.at[j]], add=True)
        plsc.subcore_barrier()

        @pl.when(sid == 0)
        def _():
            pltpu.sync_copy(hist_sh, o_hbm.at[cid])

    return _k(idx32, zeros_col, ones_col)


def _sims_body(q_ref, k_ref, s_ref, cmt_ref):
    i = pl.program_id(0)
    q = q_ref[...]
    qs = jnp.sum(q * q, axis=1, keepdims=True)
    qn = q / jnp.maximum(jnp.sqrt(qs), 1e-8)
    kb = k_ref[...]
    ks = jnp.sum(kb * kb, axis=1, keepdims=True)
    kn = kb / jnp.maximum(jnp.sqrt(ks), 1e-8)
    s = jax.lax.dot_general(
        qn, kn, (((1,), (1,)), ((), ())),
        preferred_element_type=jnp.float32,
    )
    st = jax.lax.dot_general(
        kn, qn, (((1,), (1,)), ((), ())),
        preferred_element_type=jnp.float32,
    )

    @pl.when(i < _NBLK - 1)
    def _():
        s_ref[...] = s
        cmt_ref[...] = jnp.max(st.reshape(_CPB, _W, _B), axis=1)

    @pl.when(i == _NBLK - 1)
    def _():
        lane = jax.lax.broadcasted_iota(jnp.int32, (_B, _KB), 1)
        s_ref[...] = jnp.where(i * _KB + lane < _N, s, -2.0)
        row = jax.lax.broadcasted_iota(jnp.int32, (_KB, _B), 0)
        st2 = jnp.where(i * _KB + row < _N, st, -2.0)
        cmt_ref[...] = jnp.max(st2.reshape(_CPB, _W, _B), axis=1)


def _run_sims(queries, keys_p):
    return pl.pallas_call(
        _sims_body,
        grid=(_NBLK,),
        in_specs=[
            pl.BlockSpec((_B, _D), lambda i: (0, 0)),
            pl.BlockSpec((_KB, _D), lambda i: (i, 0)),
        ],
        out_specs=[
            pl.BlockSpec((_B, _KB), lambda i: (0, i)),
            pl.BlockSpec((_CPB, _B), lambda i: (i, 0)),
        ],
        out_shape=[
            jax.ShapeDtypeStruct((_B, _NP), jnp.float32),
            jax.ShapeDtypeStruct((_C, _B), jnp.float32),
        ],
    )(queries, keys_p)


def _chunksel_body(cmt_ref, ids_ref, flat_ref):
    a = cmt_ref[...]
    row = jax.lax.broadcasted_iota(jnp.int32, (_C, _B), 0)
    qlane = jax.lax.broadcasted_iota(jnp.int32, (1, _B), 1)
    ids = []
    for t in range(_T):
        m = jnp.max(a, axis=0, keepdims=True)
        sel = jnp.where(a >= m, row, _BIGI)
        idx = jnp.min(sel, axis=0, keepdims=True)
        ids.append(idx)
        a = jnp.where(row == idx, _NEG, a)
    ids = jnp.concatenate(ids, axis=0)
    ids_ref[...] = ids
    flat_ref[...] = qlane * _C + ids


def _run_chunksel(cm_t):
    return pl.pallas_call(
        _chunksel_body,
        in_specs=[pl.BlockSpec((_C, _B), lambda: (0, 0))],
        out_specs=[
            pl.BlockSpec((_T, _B), lambda: (0, 0)),
            pl.BlockSpec((_T, _B), lambda: (0, 0)),
        ],
        out_shape=[
            jax.ShapeDtypeStruct((_T, _B), jnp.int32),
            jax.ShapeDtypeStruct((_T, _B), jnp.int32),
        ],
    )(cm_t)


_RTILE = 128
_CW = _T * _W


def _final_body(cand_ref, ids_ref, topv_ref, topi_ref, w_ref):
    a = cand_ref[...]
    selids = ids_ref[...]
    lane = jax.lax.broadcasted_iota(jnp.int32, (_RTILE, _CW), 1)
    tlane = jax.lax.broadcasted_iota(jnp.int32, (_RTILE, _T), 1)
    vals, gids = [], []
    for r in range(_KTOP):
        m = jnp.max(a, axis=1, keepdims=True)
        sel = jnp.where(a >= m, lane, _BIGI)
        pos = jnp.min(sel, axis=1, keepdims=True)
        tstar = jax.lax.shift_right_logical(pos, 7)
        wstar = jnp.bitwise_and(pos, 127)
        cid = jnp.sum(
            jnp.where(tlane == tstar, selids, 0), axis=1, keepdims=True)
        gids.append(cid * _W + wstar)
        vals.append(m)
        a = jnp.where(lane == pos, _NEG, a)
    topv = jnp.concatenate(vals, axis=1)
    topi = jnp.concatenate(gids, axis=1)
    topv_ref[...] = topv
    topi_ref[...] = topi
    e = jnp.exp(topv - topv[:, :1])
    w_ref[...] = e / jnp.sum(e, axis=1, keepdims=True)


def _run_final(cand, selids_q):
    return pl.pallas_call(
        _final_body,
        grid=(_B // _RTILE,),
        in_specs=[
            pl.BlockSpec((_RTILE, _CW), lambda i: (i, 0)),
            pl.BlockSpec((_RTILE, _T), lambda i: (i, 0)),
        ],
        out_specs=[
            pl.BlockSpec((_RTILE, _KTOP), lambda i: (i, 0)),
            pl.BlockSpec((_RTILE, _KTOP), lambda i: (i, 0)),
            pl.BlockSpec((_RTILE, _KTOP), lambda i: (i, 0)),
        ],
        out_shape=[
            jax.ShapeDtypeStruct((_B, _KTOP), jnp.float32),
            jax.ShapeDtypeStruct((_B, _KTOP), jnp.int32),
            jax.ShapeDtypeStruct((_B, _KTOP), jnp.float32),
        ],
    )(cand, selids_q)


def _combine_body(v_ref, p_ref, w_ref, o_ref):
    v = v_ref[...]
    p = p_ref[...]
    w = w_ref[...]
    acc = jnp.zeros((_RTILE, _D), jnp.float32)
    for t in range(_KTOP):
        lo = v[:, t * 128:t * 128 + _D]
        hi = v[:, t * 128 + _D:(t + 1) * 128]
        sel = jnp.where(p[:, t:t + 1] == 1, hi, lo)
        acc = acc + sel * w[:, t:t + 1]
    o_ref[...] = acc


def _run_combine(vals_g, parity, weights):
    return pl.pallas_call(
        _combine_body,
        grid=(_B // _RTILE,),
        in_specs=[
            pl.BlockSpec((_RTILE, _KTOP * 128), lambda i: (i, 0)),
            pl.BlockSpec((_RTILE, _KTOP), lambda i: (i, 0)),
            pl.BlockSpec((_RTILE, _KTOP), lambda i: (i, 0)),
        ],
        out_specs=pl.BlockSpec((_RTILE, _D), lambda i: (i, 0)),
        out_shape=jax.ShapeDtypeStruct((_B, _D), jnp.float32),
    )(vals_g, parity, weights)


_HR = _NP // 1024


def _stats_body(h_ref, c_ref, l_ref, t_ref, co_ref, lo_ref):
    h = h_ref[0] + h_ref[1]
    c = c_ref[...]
    lst = l_ref[...]
    t = t_ref[...]
    co_ref[...] = jnp.minimum(c + h, _MAXA)
    lo_ref[...] = jnp.where(h > 0, t[0, 0], lst)


def _run_stats(hist2, counts2, last2, tarr):
    return pl.pallas_call(
        _stats_body,
        in_specs=[
            pl.BlockSpec((2, _HR, 1024), lambda: (0, 0, 0)),
            pl.BlockSpec((_HR, 1024), lambda: (0, 0)),
            pl.BlockSpec((_HR, 1024), lambda: (0, 0)),
            pl.BlockSpec((8, 128), lambda: (0, 0)),
        ],
        out_specs=[
            pl.BlockSpec((_HR, 1024), lambda: (0, 0)),
            pl.BlockSpec((_HR, 1024), lambda: (0, 0)),
        ],
        out_shape=[
            jax.ShapeDtypeStruct((_HR, 1024), jnp.float32),
            jax.ShapeDtypeStruct((_HR, 1024), jnp.float32),
        ],
    )(hist2, counts2, last2, tarr)


def kernel(queries, keys_mem, values_mem, access_counts, last_access, time_step, k):
    sims, cm_t = _run_sims(queries, keys_mem)
    selids, flatrows = _run_chunksel(cm_t)

    sims_flat = sims.reshape(_B * _C, _W)
    idx_q = flatrows.T.reshape(-1)
    cand = _sc_gather(sims_flat, idx_q, _W)
    cand = cand.reshape(_B, _CW)

    selids_q = selids.T
    topv, topi, weights = _run_final(cand, selids_q)

    pair_idx = jax.lax.shift_right_logical(topi, 1)
    parity = jnp.bitwise_and(topi, 1)
    vals_g = _sc_gather(
        values_mem.reshape(_N // 2, 2 * _D), pair_idx.reshape(-1), 2 * _D)
    vals_g = vals_g.reshape(_B, _KTOP * 2 * _D)
    combined = _run_combine(vals_g, parity, weights)

    hists = _sc_hist(
        topi.reshape(32, 8, 128),
        jnp.zeros((_NP,), jnp.float32),
        jnp.ones((128,), jnp.float32),
    )

    counts2 = jnp.concatenate(
        [access_counts, jnp.zeros((_NP - _N,), jnp.float32)]).reshape(_HR, 1024)
    last2 = jnp.concatenate(
        [last_access, jnp.zeros((_NP - _N,), jnp.float32)]).reshape(_HR, 1024)
    tarr = jnp.full((8, 128), time_step, jnp.float32)
    co, lo = _run_stats(
        hists.reshape(2, _HR, 1024), counts2, last2, tarr)

    k_dep = (jnp.minimum(k, _N) - _KTOP).astype(jnp.float32)
    top_k_values = topv + k_dep
    new_access_counts = co.reshape(-1)[:_N]
    new_last_access = lo.reshape(-1)[:_N]
    return (combined, top_k_values, new_access_counts, new_last_access)
